# baseline (speedup 1.0000x reference)
import jax
import jax.numpy as jnp
from jax.experimental import pallas as pl


def _copy_kernel(x_ref, o_ref):
    o_ref[...] = x_ref[...]


def kernel(x, scale_p, scale_n):
    del scale_p, scale_n
    m, n = x.shape
    blk = 2048
    out = pl.pallas_call(
        _copy_kernel,
        grid=(n // blk,),
        in_specs=[pl.BlockSpec((m, blk), lambda j: (0, j))],
        out_specs=pl.BlockSpec((m, blk), lambda j: (0, j)),
        out_shape=jax.ShapeDtypeStruct((m, n), x.dtype),
    )(x)
    return out


# speedup vs baseline: 1.0050x; 1.0050x over previous
import jax
import jax.numpy as jnp
from jax.experimental import pallas as pl


def _copy_kernel(x_ref, o_ref):
    o_ref[...] = x_ref[...]


def kernel(x, scale_p, scale_n):
    del scale_p, scale_n
    m, n = x.shape
    blk = 8
    out = pl.pallas_call(
        _copy_kernel,
        grid=(m // blk,),
        in_specs=[pl.BlockSpec((blk, n), lambda i: (i, 0))],
        out_specs=pl.BlockSpec((blk, n), lambda i: (i, 0)),
        out_shape=jax.ShapeDtypeStruct((m, n), x.dtype),
    )(x)
    return out
